# baseline (speedup 1.0000x reference)
import functools

import jax
import jax.numpy as jnp
from jax import lax
from jax.experimental import pallas as pl
from jax.experimental.pallas import tpu as pltpu
from jax.experimental.pallas import tpu_sc as plsc

NUM_EXPERTS = 64
TOP_K = 8
BLOCK_R = 1024
N_WORKERS = 32
ROWS_PER_WORKER = 8192 // N_WORKERS


def _mm_kernel(x_ref, w_ref, o_ref):
    o_ref[...] = jax.lax.dot_general(
        x_ref[...], w_ref[...], (((1,), (1,)), ((), ())),
        preferred_element_type=jnp.float32,
    )


def _tc_logits(x, gate_weight):
    n, d = x.shape
    return pl.pallas_call(
        _mm_kernel,
        grid=(n // BLOCK_R,),
        in_specs=[
            pl.BlockSpec((BLOCK_R, d), lambda i: (i, 0)),
            pl.BlockSpec((NUM_EXPERTS, d), lambda i: (0, 0)),
        ],
        out_specs=pl.BlockSpec((BLOCK_R, NUM_EXPERTS), lambda i: (i, 0)),
        out_shape=jax.ShapeDtypeStruct((n, NUM_EXPERTS), jnp.float32),
    )(x, gate_weight)


def _merge(ka, va, kb, vb):
    kar = lax.rev(ka, (0,))
    var = lax.rev(va, (0,))
    sel = kar > kb
    hk = jnp.where(sel, kar, kb)
    hv = jnp.where(sel, var, vb)
    return plsc.sort_key_val(hk, hv, descending=True)


def _sc_topk_body(lg_hbm, ow_hbm, oi_hbm, lg_v, w_v, i_v):
    wid = lax.axis_index("s") * 2 + lax.axis_index("c")
    base = wid * ROWS_PER_WORKER
    pltpu.sync_copy(lg_hbm.at[pl.ds(base, ROWS_PER_WORKER)], lg_v)
    iota = lax.iota(jnp.int32, 16)
    lane_lo = iota < TOP_K

    def row(r, carry):
        s = [
            plsc.sort_key_val(
                lg_v[r, pl.ds(16 * j, 16)], iota + 16 * j, descending=True
            )
            for j in range(4)
        ]
        m01 = _merge(*s[0], *s[1])
        m23 = _merge(*s[2], *s[3])
        kf, vf = _merge(*m01, *m23)
        mx = jnp.max(kf)
        e = jnp.exp(kf - mx)
        em = jnp.where(lane_lo, e, 0.0)
        w = em / jnp.sum(em)
        plsc.store_compressed(w_v.at[pl.ds(r * TOP_K, 16)], w, mask=lane_lo)
        plsc.store_compressed(i_v.at[pl.ds(r * TOP_K, 16)], vf, mask=lane_lo)
        return carry

    lax.fori_loop(0, ROWS_PER_WORKER, row, 0)
    out_elems = ROWS_PER_WORKER * TOP_K
    pltpu.sync_copy(w_v.at[pl.ds(0, out_elems)],
                    ow_hbm.at[pl.ds(base * TOP_K, out_elems)])
    pltpu.sync_copy(i_v.at[pl.ds(0, out_elems)],
                    oi_hbm.at[pl.ds(base * TOP_K, out_elems)])


_sc_topk = functools.partial(
    pl.kernel,
    out_type=[
        jax.ShapeDtypeStruct((8192 * TOP_K,), jnp.float32),
        jax.ShapeDtypeStruct((8192 * TOP_K,), jnp.int32),
    ],
    mesh=plsc.VectorSubcoreMesh(
        core_axis_name="c", subcore_axis_name="s", num_cores=2, num_subcores=16
    ),
    compiler_params=pltpu.CompilerParams(needs_layout_passes=False),
    scratch_types=[
        pltpu.VMEM((ROWS_PER_WORKER, NUM_EXPERTS), jnp.float32),
        pltpu.VMEM((ROWS_PER_WORKER * TOP_K + 16,), jnp.float32),
        pltpu.VMEM((ROWS_PER_WORKER * TOP_K + 16,), jnp.int32),
    ],
)(_sc_topk_body)


def kernel(x, gate_weight):
    n, _ = x.shape
    logits = _tc_logits(x, gate_weight)
    wf, idxf = _sc_topk(logits)
    return wf.reshape(n, TOP_K), idxf.reshape(n, TOP_K)


# speedup vs baseline: 1.1610x; 1.1610x over previous
import functools

import jax
import jax.numpy as jnp
from jax import lax
from jax.experimental import pallas as pl
from jax.experimental.pallas import tpu as pltpu
from jax.experimental.pallas import tpu_sc as plsc

NUM_EXPERTS = 64
TOP_K = 8
BLOCK_R = 1024
N_WORKERS = 32
ROWS_PER_WORKER = 8192 // N_WORKERS


def _mm_kernel(x_ref, w_ref, o_ref):
    o_ref[...] = jax.lax.dot_general(
        x_ref[...], w_ref[...], (((1,), (1,)), ((), ())),
        preferred_element_type=jnp.float32,
    )


def _tc_logits(x, gate_weight):
    n, d = x.shape
    return pl.pallas_call(
        _mm_kernel,
        grid=(n // BLOCK_R,),
        in_specs=[
            pl.BlockSpec((BLOCK_R, d), lambda i: (i, 0)),
            pl.BlockSpec((NUM_EXPERTS, d), lambda i: (0, 0)),
        ],
        out_specs=pl.BlockSpec((BLOCK_R, NUM_EXPERTS), lambda i: (i, 0)),
        out_shape=jax.ShapeDtypeStruct((n, NUM_EXPERTS), jnp.float32),
    )(x, gate_weight)


def _merge(ka, va, kb, vb):
    kar = lax.rev(ka, (0,))
    var = lax.rev(va, (0,))
    sel = kar > kb
    hk = jnp.where(sel, kar, kb)
    hv = jnp.where(sel, var, vb)
    return plsc.sort_key_val(hk, hv, descending=True)


def _sc_topk_body(lg_hbm, ow_hbm, oi_hbm, lg_v, w_v, i_v):
    wid = lax.axis_index("s") * 2 + lax.axis_index("c")
    base = wid * ROWS_PER_WORKER
    pltpu.sync_copy(lg_hbm.at[pl.ds(base, ROWS_PER_WORKER)], lg_v)
    iota = lax.iota(jnp.int32, 16)
    lane_lo = iota < TOP_K

    @plsc.parallel_loop(0, ROWS_PER_WORKER, unroll=4)
    def row(r):
        s = [
            plsc.sort_key_val(
                lg_v[r, pl.ds(16 * j, 16)], iota + 16 * j, descending=True
            )
            for j in range(4)
        ]
        m01 = _merge(*s[0], *s[1])
        m23 = _merge(*s[2], *s[3])
        kf, vf = _merge(*m01, *m23)
        mx = jnp.max(kf)
        e = jnp.exp(kf - mx)
        em = jnp.where(lane_lo, e, 0.0)
        w = em / jnp.sum(em)
        plsc.store_compressed(w_v.at[pl.ds(r * TOP_K, 16)], w, mask=lane_lo)
        plsc.store_compressed(i_v.at[pl.ds(r * TOP_K, 16)], vf, mask=lane_lo)
    out_elems = ROWS_PER_WORKER * TOP_K
    pltpu.sync_copy(w_v.at[pl.ds(0, out_elems)],
                    ow_hbm.at[pl.ds(base * TOP_K, out_elems)])
    pltpu.sync_copy(i_v.at[pl.ds(0, out_elems)],
                    oi_hbm.at[pl.ds(base * TOP_K, out_elems)])


_sc_topk = functools.partial(
    pl.kernel,
    out_type=[
        jax.ShapeDtypeStruct((8192 * TOP_K,), jnp.float32),
        jax.ShapeDtypeStruct((8192 * TOP_K,), jnp.int32),
    ],
    mesh=plsc.VectorSubcoreMesh(
        core_axis_name="c", subcore_axis_name="s", num_cores=2, num_subcores=16
    ),
    compiler_params=pltpu.CompilerParams(needs_layout_passes=False),
    scratch_types=[
        pltpu.VMEM((ROWS_PER_WORKER, NUM_EXPERTS), jnp.float32),
        pltpu.VMEM((ROWS_PER_WORKER * TOP_K + 16,), jnp.float32),
        pltpu.VMEM((ROWS_PER_WORKER * TOP_K + 16,), jnp.int32),
    ],
)(_sc_topk_body)


def kernel(x, gate_weight):
    n, _ = x.shape
    logits = _tc_logits(x, gate_weight)
    wf, idxf = _sc_topk(logits)
    return wf.reshape(n, TOP_K), idxf.reshape(n, TOP_K)
